# baseline (speedup 1.0000x reference)
import jax
import jax.numpy as jnp
from jax.experimental import pallas as pl
from jax.experimental.pallas import tpu as pltpu

_M_TILE = 200


def _gc_kernel(a_ref, x_ref, w_ref, b_ref, o_ref, s_ref):
    @pl.when(pl.program_id(0) == 0)
    def _():
        s_ref[...] = jnp.dot(
            x_ref[...], w_ref[...], preferred_element_type=jnp.float32
        ).astype(jnp.bfloat16)

    o_ref[...] = (
        jnp.dot(
            a_ref[...].astype(jnp.bfloat16),
            s_ref[...],
            preferred_element_type=jnp.float32,
        )
        + b_ref[...]
    )


def kernel(adjacency, input_feature, weight, bias):
    batch, n, _ = adjacency.shape
    d_in = input_feature.shape[-1]
    d_out = weight.shape[-1]
    a2 = adjacency.reshape(n, n)
    x2 = input_feature.reshape(n, d_in)
    b2 = bias.reshape(1, d_out)

    out = pl.pallas_call(
        _gc_kernel,
        grid=(n // _M_TILE,),
        in_specs=[
            pl.BlockSpec((_M_TILE, n), lambda i: (i, 0)),
            pl.BlockSpec((n, d_in), lambda i: (0, 0)),
            pl.BlockSpec((d_in, d_out), lambda i: (0, 0)),
            pl.BlockSpec((1, d_out), lambda i: (0, 0)),
        ],
        out_specs=pl.BlockSpec((_M_TILE, d_out), lambda i: (i, 0)),
        out_shape=jax.ShapeDtypeStruct((n, d_out), jnp.float32),
        scratch_shapes=[pltpu.VMEM((n, d_out), jnp.bfloat16)],
    )(a2, x2, weight, b2)
    return out.reshape(batch, n, d_out)


# speedup vs baseline: 1.0121x; 1.0121x over previous
import jax
import jax.numpy as jnp
from jax.experimental import pallas as pl
from jax.experimental.pallas import tpu as pltpu

_M_TILE = 400


def _gc_kernel(a_ref, x_ref, w_ref, b_ref, o_ref, s_ref):
    @pl.when(pl.program_id(0) == 0)
    def _():
        s_ref[...] = jnp.dot(
            x_ref[...], w_ref[...], preferred_element_type=jnp.float32
        ).astype(jnp.bfloat16)

    o_ref[...] = (
        jnp.dot(
            a_ref[...].astype(jnp.bfloat16),
            s_ref[...],
            preferred_element_type=jnp.float32,
        )
        + b_ref[...]
    )


def kernel(adjacency, input_feature, weight, bias):
    batch, n, _ = adjacency.shape
    d_in = input_feature.shape[-1]
    d_out = weight.shape[-1]
    a2 = adjacency.reshape(n, n)
    x2 = input_feature.reshape(n, d_in)
    b2 = bias.reshape(1, d_out)

    out = pl.pallas_call(
        _gc_kernel,
        grid=(n // _M_TILE,),
        in_specs=[
            pl.BlockSpec((_M_TILE, n), lambda i: (i, 0)),
            pl.BlockSpec((n, d_in), lambda i: (0, 0)),
            pl.BlockSpec((d_in, d_out), lambda i: (0, 0)),
            pl.BlockSpec((1, d_out), lambda i: (0, 0)),
        ],
        out_specs=pl.BlockSpec((_M_TILE, d_out), lambda i: (i, 0)),
        out_shape=jax.ShapeDtypeStruct((n, d_out), jnp.float32),
        scratch_shapes=[pltpu.VMEM((n, d_out), jnp.bfloat16)],
    )(a2, x2, weight, b2)
    return out.reshape(batch, n, d_out)
